# Initial kernel scaffold; baseline (speedup 1.0000x reference)
import jax
import jax.numpy as jnp
from jax.experimental import pallas as pl


def kernel(images, predictions):
    raise NotImplementedError("write your pallas kernel here")


# speedup vs baseline: 6.4764x; 6.4764x over previous
import numpy as np
import jax
import jax.numpy as jnp
from jax import lax
from jax.experimental import pallas as pl

NUM_CLASSES = 80
KSEL = 1000
MAXPC = 10
MAXTOT = 10
IOU_T = 0.5
SCORE_T = 0.05
LANES = 128
NEG = -1e30


def _make_anchors(image_size):
    aspect_ratios = [0.5, 1.0, 2.0]
    scales = [2.0 ** 0, 2.0 ** (1.0 / 3.0), 2.0 ** (2.0 / 3.0)]
    step = int((512 - 32) / 4)
    areas = [(x * step + 32) ** 2 for x in range(5)]
    strides = [2 ** i for i in range(3, 8)]
    out = []
    for li in range(5):
        area = float(areas[li])
        dims = []
        for ratio in aspect_ratios:
            h = np.sqrt(area / ratio)
            w = area / h
            for scale in scales:
                dims.append([scale * w, scale * h])
        dims = np.array(dims, dtype=np.float32)
        fs = int(np.ceil(image_size / strides[li]))
        rx = np.arange(fs, dtype=np.float32) + 0.5
        ry = np.arange(fs, dtype=np.float32) + 0.5
        xx, yy = np.meshgrid(rx, ry)
        centers = np.stack([xx, yy], axis=-1) * strides[li]
        centers = np.tile(centers[:, :, None, :], (1, 1, 9, 1))
        dimsT = np.tile(dims[None, None, :, :], (fs, fs, 1, 1))
        anchors = np.concatenate([centers, dimsT], axis=-1).reshape(-1, 4)
        out.append(anchors)
    return np.concatenate(out, axis=0).astype(np.float32)


_ANCH = _make_anchors(512)
N_REAL = _ANCH.shape[0]
NPAD = ((N_REAL + 8 * LANES - 1) // (8 * LANES)) * (8 * LANES)
NB = NPAD // LANES

_APL = np.zeros((4, NPAD), dtype=np.float32)
_APL[:, :N_REAL] = _ANCH.T
_APL[2:, N_REAL:] = 1.0
_APL = _APL.reshape(4, NB, LANES)


def _decode_body(p_ref, a_ref, o_ref):
    tx = p_ref[0, 0]
    ty = p_ref[0, 1]
    tw = p_ref[0, 2]
    th = p_ref[0, 3]
    acx = a_ref[0]
    acy = a_ref[1]
    aw = a_ref[2]
    ah = a_ref[3]
    cx = tx * 0.1 * aw + acx
    cy = ty * 0.1 * ah + acy
    w = jnp.exp(tw * 0.2) * aw
    h = jnp.exp(th * 0.2) * ah
    o_ref[0, 0] = cx - w / 2.0
    o_ref[0, 1] = cy - h / 2.0
    o_ref[0, 2] = cx + w / 2.0
    o_ref[0, 3] = cy + h / 2.0


def _nms_body(lg_ref, bx_ref, o_ref):
    lg = lg_ref[0, 0]
    sc = 1.0 / (1.0 + jnp.exp(-lg))
    x1 = bx_ref[0, 0]
    y1 = bx_ref[0, 1]
    x2 = bx_ref[0, 2]
    y2 = bx_ref[0, 3]

    u = lax.bitcast_convert_type(sc, jnp.int32)

    def sbody(i, base):
        cand = base | (jnp.int32(1) << (jnp.int32(30) - i))
        cnt = jnp.sum((u >= cand).astype(jnp.int32))
        return jnp.where(cnt >= KSEL, cand, base)

    tstar = lax.fori_loop(0, 31, sbody, jnp.int32(0))

    work = jnp.where(u >= tstar, sc, NEG)

    r_io = lax.broadcasted_iota(jnp.int32, (NB, LANES), 0)
    l_io = lax.broadcasted_iota(jnp.int32, (NB, LANES), 1)
    fid = r_io * LANES + l_io
    a2 = jnp.maximum(x2 - x1, 0.0) * jnp.maximum(y2 - y1, 0.0)

    o_r = lax.broadcasted_iota(jnp.int32, (8, LANES), 0)
    o_l = lax.broadcasted_iota(jnp.int32, (8, LANES), 1)
    acc = jnp.zeros((8, LANES), jnp.float32)

    for i in range(MAXPC):
        m = jnp.max(work)
        bf = jnp.min(jnp.where(work == m, fid, jnp.int32(NPAD)))
        selm = fid == bf
        bx1 = jnp.sum(jnp.where(selm, x1, 0.0))
        by1 = jnp.sum(jnp.where(selm, y1, 0.0))
        bx2 = jnp.sum(jnp.where(selm, x2, 0.0))
        by2 = jnp.sum(jnp.where(selm, y2, 0.0))
        valid = m >= SCORE_T

        def put(r, v):
            return jnp.where((o_r == r) & (o_l == i), v, 0.0)

        acc = (acc + put(0, jnp.where(valid, bx1, 0.0))
               + put(1, jnp.where(valid, by1, 0.0))
               + put(2, jnp.where(valid, bx2, 0.0))
               + put(3, jnp.where(valid, by2, 0.0))
               + put(4, jnp.where(valid, m, 0.0)))

        ix1 = jnp.maximum(bx1, x1)
        iy1 = jnp.maximum(by1, y1)
        ix2 = jnp.minimum(bx2, x2)
        iy2 = jnp.minimum(by2, y2)
        inter = jnp.maximum(ix2 - ix1, 0.0) * jnp.maximum(iy2 - iy1, 0.0)
        ba = jnp.maximum(bx2 - bx1, 0.0) * jnp.maximum(by2 - by1, 0.0)
        union = ba + a2 - inter
        iou = jnp.where(union > 0.0, inter / jnp.maximum(union, 1e-8), 0.0)
        work = jnp.where((iou > IOU_T) | selm, NEG, work)

    o_ref[0, 0] = acc


def _merge_body(i_ref, o_ref):
    arr = i_ref[0].reshape(NUM_CLASSES * 8, LANES)
    rr = lax.broadcasted_iota(jnp.int32, (NUM_CLASSES * 8, LANES), 0)
    ll = lax.broadcasted_iota(jnp.int32, (NUM_CLASSES * 8, LANES), 1)
    c_io = rr >> 3
    r_io = rr & 7
    score_reg = (r_io == 4) & (ll < MAXPC)
    sl = jnp.where(score_reg, arr, NEG)
    flat = c_io * MAXPC + ll

    o_r = lax.broadcasted_iota(jnp.int32, (8, LANES), 0)
    o_l = lax.broadcasted_iota(jnp.int32, (8, LANES), 1)
    acc = jnp.zeros((8, LANES), jnp.float32)
    nval = jnp.float32(0.0)

    for i in range(MAXTOT):
        m = jnp.max(sl)
        bf = jnp.min(jnp.where(sl == m, flat, jnp.int32(NUM_CLASSES * MAXPC)))
        ci = bf // MAXPC
        li = bf - ci * MAXPC
        selc = (c_io == ci) & (ll == li)
        gx1 = jnp.sum(jnp.where(selc & (r_io == 0), arr, 0.0))
        gy1 = jnp.sum(jnp.where(selc & (r_io == 1), arr, 0.0))
        gx2 = jnp.sum(jnp.where(selc & (r_io == 2), arr, 0.0))
        gy2 = jnp.sum(jnp.where(selc & (r_io == 3), arr, 0.0))
        valid = m > 0.0

        def put(r, v):
            return jnp.where((o_r == r) & (o_l == i), v, 0.0)

        acc = (acc + put(0, jnp.where(valid, gx1, 0.0))
               + put(1, jnp.where(valid, gy1, 0.0))
               + put(2, jnp.where(valid, gx2, 0.0))
               + put(3, jnp.where(valid, gy2, 0.0))
               + put(4, jnp.where(valid, m, 0.0))
               + put(5, jnp.where(valid, ci.astype(jnp.float32), 0.0)))
        nval = nval + jnp.where(valid, 1.0, 0.0)
        sl = jnp.where(selc & (r_io == 4), NEG, sl)

    acc = acc + jnp.where((o_r == 6) & (o_l == 0), nval, 0.0)
    o_ref[0] = acc


def kernel(images, predictions):
    del images
    B = predictions.shape[0]
    padn = NPAD - N_REAL

    predT = jnp.transpose(predictions[:, :, :4], (0, 2, 1))
    predT = jnp.pad(predT, ((0, 0), (0, 0), (0, padn)))
    predT = predT.reshape(B, 4, NB, LANES)

    lg = jnp.transpose(predictions[:, :, 4:], (0, 2, 1))
    lg = jnp.pad(lg, ((0, 0), (0, 0), (0, padn)), constant_values=NEG)
    lg = lg.reshape(B, NUM_CLASSES, NB, LANES)

    anch = jnp.asarray(_APL)

    boxes = pl.pallas_call(
        _decode_body,
        grid=(B,),
        in_specs=[
            pl.BlockSpec((1, 4, NB, LANES), lambda b: (b, 0, 0, 0)),
            pl.BlockSpec((4, NB, LANES), lambda b: (0, 0, 0)),
        ],
        out_specs=pl.BlockSpec((1, 4, NB, LANES), lambda b: (b, 0, 0, 0)),
        out_shape=jax.ShapeDtypeStruct((B, 4, NB, LANES), jnp.float32),
    )(predT, anch)

    per_class = pl.pallas_call(
        _nms_body,
        grid=(B, NUM_CLASSES),
        in_specs=[
            pl.BlockSpec((1, 1, NB, LANES), lambda b, c: (b, c, 0, 0)),
            pl.BlockSpec((1, 4, NB, LANES), lambda b, c: (b, 0, 0, 0)),
        ],
        out_specs=pl.BlockSpec((1, 1, 8, LANES), lambda b, c: (b, c, 0, 0)),
        out_shape=jax.ShapeDtypeStruct((B, NUM_CLASSES, 8, LANES), jnp.float32),
    )(lg, boxes)

    out3 = pl.pallas_call(
        _merge_body,
        grid=(B,),
        in_specs=[
            pl.BlockSpec((1, NUM_CLASSES, 8, LANES), lambda b: (b, 0, 0, 0)),
        ],
        out_specs=pl.BlockSpec((1, 8, LANES), lambda b: (b, 0, 0)),
        out_shape=jax.ShapeDtypeStruct((B, 8, LANES), jnp.float32),
    )(per_class)

    nb = jnp.stack(
        [out3[:, 0, :MAXTOT], out3[:, 1, :MAXTOT],
         out3[:, 2, :MAXTOT], out3[:, 3, :MAXTOT]], axis=-1)
    ns = out3[:, 4, :MAXTOT]
    nc = out3[:, 5, :MAXTOT]
    nv = out3[:, 6, 0].astype(jnp.int32)
    return nb, ns, nc, nv


# speedup vs baseline: 7.5202x; 1.1612x over previous
import functools

import numpy as np
import jax
import jax.numpy as jnp
from jax import lax
from jax.experimental import pallas as pl
from jax.experimental.pallas import tpu as pltpu
from jax.experimental.pallas import tpu_sc as plsc

NUM_CLASSES = 80
KSEL = 1000
MAXPC = 10
MAXTOT = 10
IOU_T = 0.5
SCORE_T = 0.05
LANES = 128
NEG = -1e30
CAP = 2048
SIG_CH = 8192
SC_CHUNK = 8192
PAIRS_PER_TILE = 10


def _make_anchors(image_size):
    aspect_ratios = [0.5, 1.0, 2.0]
    scales = [2.0 ** 0, 2.0 ** (1.0 / 3.0), 2.0 ** (2.0 / 3.0)]
    step = int((512 - 32) / 4)
    areas = [(x * step + 32) ** 2 for x in range(5)]
    strides = [2 ** i for i in range(3, 8)]
    out = []
    for li in range(5):
        area = float(areas[li])
        dims = []
        for ratio in aspect_ratios:
            h = np.sqrt(area / ratio)
            w = area / h
            for scale in scales:
                dims.append([scale * w, scale * h])
        dims = np.array(dims, dtype=np.float32)
        fs = int(np.ceil(image_size / strides[li]))
        rx = np.arange(fs, dtype=np.float32) + 0.5
        ry = np.arange(fs, dtype=np.float32) + 0.5
        xx, yy = np.meshgrid(rx, ry)
        centers = np.stack([xx, yy], axis=-1) * strides[li]
        centers = np.tile(centers[:, :, None, :], (1, 1, 9, 1))
        dimsT = np.tile(dims[None, None, :, :], (fs, fs, 1, 1))
        anchors = np.concatenate([centers, dimsT], axis=-1).reshape(-1, 4)
        out.append(anchors)
    return np.concatenate(out, axis=0).astype(np.float32)


_ANCH = _make_anchors(512)
N_REAL = _ANCH.shape[0]
NPAD = ((N_REAL + 8 * LANES - 1) // (8 * LANES)) * (8 * LANES)
NB = NPAD // LANES

_APL = np.zeros((4, NPAD), dtype=np.float32)
_APL[:, :N_REAL] = _ANCH.T
_APL[2:, N_REAL:] = 1.0
_APL = _APL.reshape(4, NB, LANES)


def _decode_body(p_ref, a_ref, o_ref):
    tx = p_ref[0, 0]
    ty = p_ref[0, 1]
    tw = p_ref[0, 2]
    th = p_ref[0, 3]
    acx = a_ref[0]
    acy = a_ref[1]
    aw = a_ref[2]
    ah = a_ref[3]
    cx = tx * 0.1 * aw + acx
    cy = ty * 0.1 * ah + acy
    w = jnp.exp(tw * 0.2) * aw
    h = jnp.exp(th * 0.2) * ah
    o_ref[0, 0] = cx - w / 2.0
    o_ref[0, 1] = cy - h / 2.0
    o_ref[0, 2] = cx + w / 2.0
    o_ref[0, 3] = cy + h / 2.0


def _sig_body(p_ref, s_ref, h_ref):
    x = p_ref[0]
    s = 1.0 / (1.0 + jnp.exp(-x))
    s_ref[0] = s
    h_ref[0] = s.astype(jnp.bfloat16)


def _search_body(h_ref, t_ref):
    sb = h_ref[0]
    u = lax.bitcast_convert_type(sb, jnp.int16)

    def sbody(i, base):
        cand = base | (jnp.int32(1) << (jnp.int32(14) - i))
        cand16 = cand.astype(jnp.int16)
        cnt = jnp.sum((u >= cand16).astype(jnp.int32), axis=0, keepdims=True)
        return jnp.where(cnt >= KSEL, cand, base)

    base = lax.fori_loop(0, 15, sbody, jnp.zeros((1, LANES), jnp.int32))
    tb = jnp.maximum(base - 1, 0)
    tf = lax.bitcast_convert_type(tb.astype(jnp.int16), jnp.bfloat16)
    t_ref[0] = tf.astype(jnp.float32)


def _compact_body(scm, thr, brow, os_out, ob_out,
                  thr_v, buf, cval, cidx, brows, sem):
    wid = lax.axis_index("s") * 2 + lax.axis_index("c")
    pltpu.sync_copy(thr, thr_v)

    def pair_loop(r, carry):
        p = wid * PAIRS_PER_TILE + r
        b = p // NUM_CLASSES
        grp = (p // 16) * 16
        tv = thr_v[pl.ds(grp, 16)]
        ts = jnp.max(jnp.where(lax.iota(jnp.int32, 16) == (p - grp), tv, NEG))

        def zbody(i, c2):
            cval[pl.ds(i * 16, 16)] = jnp.zeros((16,), jnp.float32)
            cidx[pl.ds(i * 16, 16)] = jnp.zeros((16,), jnp.int32)
            return c2

        lax.fori_loop(0, CAP // 16, zbody, 0)

        def chunk_body(k, off):
            pltpu.sync_copy(scm.at[pl.ds(p * NPAD + k * SC_CHUNK, SC_CHUNK)],
                            buf)

            def step(j, off2):
                v = buf[pl.ds(j * 16, 16)]
                mask = v >= ts
                nbase = b * NPAD + k * SC_CHUNK + j * 16
                iv = nbase + lax.iota(jnp.int32, 16)
                co = jnp.minimum(off2, CAP - 16)
                plsc.store_compressed(cval.at[pl.ds(co, 16)], v, mask=mask)
                plsc.store_compressed(cidx.at[pl.ds(co, 16)], iv, mask=mask)
                pc = plsc.all_reduce_population_count(mask)
                return off2 + jnp.max(pc)

            return lax.fori_loop(0, SC_CHUNK // 16, step, off)

        lax.fori_loop(0, NPAD // SC_CHUNK, chunk_body, jnp.int32(0))

        copies = []
        for g in range(CAP // 128):
            copies.append(pltpu.async_copy(
                brow.at[cidx.at[pl.ds(g * 128, 128)]],
                brows.at[pl.ds(g * 128, 128)], sem))
        for cpy in copies:
            cpy.wait()

        pltpu.sync_copy(cval, os_out.at[pl.ds(p * CAP, CAP)])
        pltpu.sync_copy(brows, ob_out.at[pl.ds(p * CAP, CAP)])
        return carry

    lax.fori_loop(0, PAIRS_PER_TILE, pair_loop, 0)


def _cnms_body(cs_ref, bx_ref, o_ref):
    sc = cs_ref[0]
    x1 = bx_ref[0, 0]
    y1 = bx_ref[0, 1]
    x2 = bx_ref[0, 2]
    y2 = bx_ref[0, 3]
    u = lax.bitcast_convert_type(sc, jnp.int32)

    def rbody(i, base):
        cand = base | (jnp.int32(1) << (jnp.int32(30) - i))
        cnt = jnp.sum((u >= cand).astype(jnp.int32), axis=0, keepdims=True)
        return jnp.where(cnt >= KSEL, cand, base)

    tstar = lax.fori_loop(0, 31, rbody, jnp.zeros((1, LANES), jnp.int32))
    work = jnp.where(u >= tstar, sc, NEG)

    r_io = lax.broadcasted_iota(jnp.int32, (CAP, LANES), 0)
    a2 = jnp.maximum(x2 - x1, 0.0) * jnp.maximum(y2 - y1, 0.0)
    o_r = lax.broadcasted_iota(jnp.int32, (56, LANES), 0)
    acc = jnp.zeros((56, LANES), jnp.float32)

    for i in range(MAXPC):
        m = jnp.max(work, axis=0, keepdims=True)
        bidx = jnp.min(jnp.where(work == m, r_io, CAP), axis=0, keepdims=True)
        selm = r_io == bidx
        bx1 = jnp.sum(jnp.where(selm, x1, 0.0), axis=0, keepdims=True)
        by1 = jnp.sum(jnp.where(selm, y1, 0.0), axis=0, keepdims=True)
        bx2 = jnp.sum(jnp.where(selm, x2, 0.0), axis=0, keepdims=True)
        by2 = jnp.sum(jnp.where(selm, y2, 0.0), axis=0, keepdims=True)
        valid = m >= SCORE_T

        def put(r, v):
            return jnp.where(o_r == r, v, 0.0)

        acc = (acc + put(5 * i + 0, jnp.where(valid, bx1, 0.0))
               + put(5 * i + 1, jnp.where(valid, by1, 0.0))
               + put(5 * i + 2, jnp.where(valid, bx2, 0.0))
               + put(5 * i + 3, jnp.where(valid, by2, 0.0))
               + put(5 * i + 4, jnp.where(valid, m, 0.0)))

        ix1 = jnp.maximum(bx1, x1)
        iy1 = jnp.maximum(by1, y1)
        ix2 = jnp.minimum(bx2, x2)
        iy2 = jnp.minimum(by2, y2)
        inter = jnp.maximum(ix2 - ix1, 0.0) * jnp.maximum(iy2 - iy1, 0.0)
        ba = jnp.maximum(bx2 - bx1, 0.0) * jnp.maximum(by2 - by1, 0.0)
        union = ba + a2 - inter
        iou = jnp.where(union > 0.0, inter / jnp.maximum(union, 1e-8), 0.0)
        work = jnp.where((iou > IOU_T) | selm, NEG, work)

    o_ref[0] = acc


def _merge_body(i_ref, o_ref):
    arr = i_ref[0]
    rr = lax.broadcasted_iota(jnp.int32, (56, LANES), 0)
    ll = lax.broadcasted_iota(jnp.int32, (56, LANES), 1)
    pick_i = rr // 5
    r_in = rr - pick_i * 5
    score_reg = (r_in == 4) & (rr < 5 * MAXPC) & (ll < NUM_CLASSES)
    sl = jnp.where(score_reg, arr, NEG)
    flat = ll * MAXPC + pick_i

    o_r = lax.broadcasted_iota(jnp.int32, (8, LANES), 0)
    o_l = lax.broadcasted_iota(jnp.int32, (8, LANES), 1)
    acc = jnp.zeros((8, LANES), jnp.float32)
    nval = jnp.float32(0.0)

    for i in range(MAXTOT):
        m = jnp.max(sl)
        bf = jnp.min(jnp.where(sl == m, flat, jnp.int32(NUM_CLASSES * MAXPC)))
        ci = bf // MAXPC
        li = bf - ci * MAXPC
        selc = (ll == ci) & (pick_i == li) & (rr < 5 * MAXPC)
        gx1 = jnp.sum(jnp.where(selc & (r_in == 0), arr, 0.0))
        gy1 = jnp.sum(jnp.where(selc & (r_in == 1), arr, 0.0))
        gx2 = jnp.sum(jnp.where(selc & (r_in == 2), arr, 0.0))
        gy2 = jnp.sum(jnp.where(selc & (r_in == 3), arr, 0.0))
        valid = m > 0.0

        def put(r, v):
            return jnp.where((o_r == r) & (o_l == i), v, 0.0)

        acc = (acc + put(0, jnp.where(valid, gx1, 0.0))
               + put(1, jnp.where(valid, gy1, 0.0))
               + put(2, jnp.where(valid, gx2, 0.0))
               + put(3, jnp.where(valid, gy2, 0.0))
               + put(4, jnp.where(valid, m, 0.0))
               + put(5, jnp.where(valid, ci.astype(jnp.float32), 0.0)))
        nval = nval + jnp.where(valid, 1.0, 0.0)
        sl = jnp.where(selc & (r_in == 4), NEG, sl)

    acc = acc + jnp.where((o_r == 6) & (o_l == 0), nval, 0.0)
    o_ref[0] = acc


def kernel(images, predictions):
    del images
    B = predictions.shape[0]
    padn = NPAD - N_REAL
    BC = B * NUM_CLASSES

    predT = jnp.transpose(predictions[:, :, :4], (0, 2, 1))
    predT = jnp.pad(predT, ((0, 0), (0, 0), (0, padn)))
    predT = predT.reshape(B, 4, NB, LANES)
    anch = jnp.asarray(_APL)

    boxes = pl.pallas_call(
        _decode_body,
        grid=(B,),
        in_specs=[
            pl.BlockSpec((1, 4, NB, LANES), lambda b: (b, 0, 0, 0)),
            pl.BlockSpec((4, NB, LANES), lambda b: (0, 0, 0)),
        ],
        out_specs=pl.BlockSpec((1, 4, NB, LANES), lambda b: (b, 0, 0, 0)),
        out_shape=jax.ShapeDtypeStruct((B, 4, NB, LANES), jnp.float32),
    )(predT, anch)

    predp = jnp.pad(predictions, ((0, 0), (0, padn), (0, LANES - 84)),
                    constant_values=NEG)

    scol, sb16 = pl.pallas_call(
        _sig_body,
        grid=(B, NPAD // SIG_CH),
        in_specs=[pl.BlockSpec((1, SIG_CH, LANES), lambda b, k: (b, k, 0))],
        out_specs=[
            pl.BlockSpec((1, SIG_CH, LANES), lambda b, k: (b, k, 0)),
            pl.BlockSpec((1, SIG_CH, LANES), lambda b, k: (b, k, 0)),
        ],
        out_shape=[
            jax.ShapeDtypeStruct((B, NPAD, LANES), jnp.float32),
            jax.ShapeDtypeStruct((B, NPAD, LANES), jnp.bfloat16),
        ],
    )(predp)

    thr = pl.pallas_call(
        _search_body,
        grid=(B,),
        in_specs=[pl.BlockSpec((1, NPAD, LANES), lambda b: (b, 0, 0))],
        out_specs=pl.BlockSpec((1, 1, LANES), lambda b: (b, 0, 0)),
        out_shape=jax.ShapeDtypeStruct((B, 1, LANES), jnp.float32),
    )(sb16)

    scm = jnp.transpose(scol[:, :, 4:4 + NUM_CLASSES], (0, 2, 1)).reshape(-1)
    thr_flat = thr[:, 0, 4:4 + NUM_CLASSES].reshape(BC)
    brow = jnp.transpose(boxes, (0, 2, 3, 1)).reshape(B, NPAD, 4)
    brow = jnp.pad(brow, ((0, 0), (0, 0), (0, 4))).reshape(B * NPAD, 8)

    mesh = plsc.VectorSubcoreMesh(core_axis_name="c", subcore_axis_name="s")
    sc_compact = pl.kernel(
        _compact_body,
        out_type=[
            jax.ShapeDtypeStruct((BC * CAP,), jnp.float32),
            jax.ShapeDtypeStruct((BC * CAP, 8), jnp.float32),
        ],
        mesh=mesh,
        compiler_params=pltpu.CompilerParams(needs_layout_passes=False,
                                             use_tc_tiling_on_sc=False),
        scratch_types=[
            pltpu.VMEM((BC,), jnp.float32),
            pltpu.VMEM((SC_CHUNK,), jnp.float32),
            pltpu.VMEM((CAP,), jnp.float32),
            pltpu.VMEM((CAP,), jnp.int32),
            pltpu.VMEM((CAP, 8), jnp.float32),
            pltpu.SemaphoreType.DMA,
        ],
    )
    os_, ob_ = sc_compact(scm, thr_flat, brow)

    cs_col = os_.reshape(B, NUM_CLASSES, CAP).transpose(0, 2, 1)
    cs_col = jnp.pad(cs_col, ((0, 0), (0, 0), (0, LANES - NUM_CLASSES)))
    cb = ob_.reshape(B, NUM_CLASSES, CAP, 8)[:, :, :, :4]
    cb = cb.transpose(0, 3, 2, 1)
    cb = jnp.pad(cb, ((0, 0), (0, 0), (0, 0), (0, LANES - NUM_CLASSES)))

    out56 = pl.pallas_call(
        _cnms_body,
        grid=(B,),
        in_specs=[
            pl.BlockSpec((1, CAP, LANES), lambda b: (b, 0, 0)),
            pl.BlockSpec((1, 4, CAP, LANES), lambda b: (b, 0, 0, 0)),
        ],
        out_specs=pl.BlockSpec((1, 56, LANES), lambda b: (b, 0, 0)),
        out_shape=jax.ShapeDtypeStruct((B, 56, LANES), jnp.float32),
    )(cs_col, cb)

    out3 = pl.pallas_call(
        _merge_body,
        grid=(B,),
        in_specs=[pl.BlockSpec((1, 56, LANES), lambda b: (b, 0, 0))],
        out_specs=pl.BlockSpec((1, 8, LANES), lambda b: (b, 0, 0)),
        out_shape=jax.ShapeDtypeStruct((B, 8, LANES), jnp.float32),
    )(out56)

    nb = jnp.stack(
        [out3[:, 0, :MAXTOT], out3[:, 1, :MAXTOT],
         out3[:, 2, :MAXTOT], out3[:, 3, :MAXTOT]], axis=-1)
    ns = out3[:, 4, :MAXTOT]
    nc = out3[:, 5, :MAXTOT]
    nv = out3[:, 6, 0].astype(jnp.int32)
    return nb, ns, nc, nv


# speedup vs baseline: 8.6907x; 1.1556x over previous
import functools

import numpy as np
import jax
import jax.numpy as jnp
from jax import lax
from jax.experimental import pallas as pl
from jax.experimental.pallas import tpu as pltpu
from jax.experimental.pallas import tpu_sc as plsc

NUM_CLASSES = 80
KSEL = 1000
MAXPC = 10
MAXTOT = 10
IOU_T = 0.5
SCORE_T = 0.05
LANES = 128
NEG = -1e30
CAP = 2048
CROWS = CAP // LANES
CG = 16
SIG_CH = 2048
SC_CHUNK = 8192
PAIRS_PER_TILE = 10


def _make_anchors(image_size):
    aspect_ratios = [0.5, 1.0, 2.0]
    scales = [2.0 ** 0, 2.0 ** (1.0 / 3.0), 2.0 ** (2.0 / 3.0)]
    step = int((512 - 32) / 4)
    areas = [(x * step + 32) ** 2 for x in range(5)]
    strides = [2 ** i for i in range(3, 8)]
    out = []
    for li in range(5):
        area = float(areas[li])
        dims = []
        for ratio in aspect_ratios:
            h = np.sqrt(area / ratio)
            w = area / h
            for scale in scales:
                dims.append([scale * w, scale * h])
        dims = np.array(dims, dtype=np.float32)
        fs = int(np.ceil(image_size / strides[li]))
        rx = np.arange(fs, dtype=np.float32) + 0.5
        ry = np.arange(fs, dtype=np.float32) + 0.5
        xx, yy = np.meshgrid(rx, ry)
        centers = np.stack([xx, yy], axis=-1) * strides[li]
        centers = np.tile(centers[:, :, None, :], (1, 1, 9, 1))
        dimsT = np.tile(dims[None, None, :, :], (fs, fs, 1, 1))
        anchors = np.concatenate([centers, dimsT], axis=-1).reshape(-1, 4)
        out.append(anchors)
    return np.concatenate(out, axis=0).astype(np.float32)


_ANCH = _make_anchors(512)
N_REAL = _ANCH.shape[0]
NPAD = ((N_REAL + 8 * LANES - 1) // (8 * LANES)) * (8 * LANES)
NB = NPAD // LANES

_APL = np.zeros((4, NPAD), dtype=np.float32)
_APL[:, :N_REAL] = _ANCH.T
_APL[2:, N_REAL:] = 1.0
_APL = _APL.reshape(4, NB, LANES)


def _decode_body(p_ref, a_ref, o_ref):
    tx = p_ref[0, 0]
    ty = p_ref[0, 1]
    tw = p_ref[0, 2]
    th = p_ref[0, 3]
    acx = a_ref[0]
    acy = a_ref[1]
    aw = a_ref[2]
    ah = a_ref[3]
    cx = tx * 0.1 * aw + acx
    cy = ty * 0.1 * ah + acy
    w = jnp.exp(tw * 0.2) * aw
    h = jnp.exp(th * 0.2) * ah
    o_ref[0, 0] = cx - w / 2.0
    o_ref[0, 1] = cy - h / 2.0
    o_ref[0, 2] = cx + w / 2.0
    o_ref[0, 3] = cy + h / 2.0


def _sig_body(p_ref, t_ref, h_ref, s_ref):
    x = p_ref[0]
    s = 1.0 / (1.0 + jnp.exp(-x))
    t_ref[0] = jnp.transpose(s)
    h_ref[0] = s.astype(jnp.bfloat16)
    s_ref[0] = s


def _bcount_body(s_ref, t_ref, o_ref):
    s = s_ref[0]
    t = t_ref[0]
    m = (s >= t).astype(jnp.float32)
    bc = jnp.sum(m.reshape(NPAD // 16, 16, LANES), axis=1)
    o_ref[0] = jnp.transpose(bc)


def _search_body(h_ref, t_ref):
    sb = h_ref[0]
    u = lax.bitcast_convert_type(sb, jnp.int16)

    def sbody(i, base):
        cand = base | (jnp.int32(1) << (jnp.int32(14) - i))
        cand16 = cand.astype(jnp.int16)
        cnt = jnp.sum((u >= cand16).astype(jnp.int32), axis=0, keepdims=True)
        return jnp.where(cnt >= KSEL, cand, base)

    base = lax.fori_loop(0, 15, sbody, jnp.zeros((1, LANES), jnp.int32))
    tb = jnp.maximum(base - 1, 0)
    tf = lax.bitcast_convert_type(tb.astype(jnp.int16), jnp.bfloat16)
    t_ref[0] = tf.astype(jnp.float32)


def _compact_body(scm, thr, bcm, brow, os_out, op_out,
                  thr_v, buf, bcv, cval, cidx, brows, cpl, sem):
    wid = lax.axis_index("s") * 2 + lax.axis_index("c")
    pltpu.sync_copy(thr, thr_v)

    def pair_loop(r, carry):
        p = wid * PAIRS_PER_TILE + r
        b = p // NUM_CLASSES
        c = p - b * NUM_CLASSES
        grp = (p // 16) * 16
        tv = thr_v[pl.ds(grp, 16)]
        ts = jnp.max(jnp.where(lax.iota(jnp.int32, 16) == (p - grp), tv, NEG))

        def zbody(i, c2):
            cval[pl.ds(i * 16, 16)] = jnp.zeros((16,), jnp.float32)
            cidx[pl.ds(i * 16, 16)] = jnp.zeros((16,), jnp.int32)
            return c2

        lax.fori_loop(0, CAP // 16, zbody, 0)
        row = b * LANES + 4 + c
        pltpu.sync_copy(scm.at[pl.ds(row * NPAD, NPAD)], buf)
        pltpu.sync_copy(bcm.at[pl.ds(row * (NPAD // 16), NPAD // 16)], bcv)

        def grp_body(g, off):
            cfs = bcv[pl.ds(g * 16, 16)].astype(jnp.int32)
            for q in range(16):
                cf = cfs[q]

                def write(off2=off, q=q):
                    j = g * 16 + q
                    v = buf[pl.ds(j * 16, 16)]
                    mask = v >= ts
                    co = jnp.minimum(off2, CAP - 16)
                    iv = b * NPAD + j * 16 + lax.iota(jnp.int32, 16)
                    plsc.store_compressed(cval.at[pl.ds(co, 16)], v,
                                          mask=mask)
                    plsc.store_compressed(cidx.at[pl.ds(co, 16)], iv,
                                          mask=mask)

                pl.when(cf > 0)(write)
                off = off + cf
            return off

        lax.fori_loop(0, NPAD // 256, grp_body, jnp.int32(0))

        copies = []
        for g in range(CAP // 128):
            copies.append(pltpu.async_copy(
                brow.at[cidx.at[pl.ds(g * 128, 128)]],
                brows.at[pl.ds(g * 128, 128)], sem))
        for cpy in copies:
            cpy.wait()

        def gbody(g, c2):
            rows = g * 16 + lax.iota(jnp.int32, 16)
            for j in range(4):
                vals = plsc.load_gather(
                    brows, [rows, jnp.full((16,), j, jnp.int32)])
                cpl[j, pl.ds(g * 16, 16)] = vals
            return c2

        lax.fori_loop(0, CAP // 16, gbody, 0)

        pltpu.sync_copy(cval, os_out.at[pl.ds(p * CAP, CAP)])
        for j in range(4):
            pltpu.sync_copy(cpl.at[j], op_out.at[j, pl.ds(p * CAP, CAP)])
        return carry

    lax.fori_loop(0, PAIRS_PER_TILE, pair_loop, 0)


def _cnms_body(cs_ref, bx_ref, o_ref):
    sc = cs_ref[0]
    x1 = bx_ref[0, 0]
    y1 = bx_ref[1, 0]
    x2 = bx_ref[2, 0]
    y2 = bx_ref[3, 0]
    u = lax.bitcast_convert_type(sc, jnp.int32)

    def rbody(i, base):
        cand = base | (jnp.int32(1) << (jnp.int32(30) - i))
        cnt = jnp.sum((u >= cand).astype(jnp.int32), axis=(1, 2),
                      keepdims=True)
        return jnp.where(cnt >= KSEL, cand, base)

    tstar = lax.fori_loop(0, 31, rbody, jnp.zeros((CG, 1, 1), jnp.int32))
    work = jnp.where(u >= tstar, sc, NEG)

    r_io = lax.broadcasted_iota(jnp.int32, (CG, CROWS, LANES), 1)
    l_io = lax.broadcasted_iota(jnp.int32, (CG, CROWS, LANES), 2)
    fid = r_io * LANES + l_io
    a2 = jnp.maximum(x2 - x1, 0.0) * jnp.maximum(y2 - y1, 0.0)

    o_r = lax.broadcasted_iota(jnp.int32, (CG, 8, LANES), 1)
    o_l = lax.broadcasted_iota(jnp.int32, (CG, 8, LANES), 2)
    acc = jnp.zeros((CG, 8, LANES), jnp.float32)

    for i in range(MAXPC):
        m = jnp.max(work, axis=(1, 2), keepdims=True)
        bf = jnp.min(jnp.where(work == m, fid, CAP), axis=(1, 2),
                     keepdims=True)
        selm = fid == bf
        bx1 = jnp.sum(jnp.where(selm, x1, 0.0), axis=(1, 2), keepdims=True)
        by1 = jnp.sum(jnp.where(selm, y1, 0.0), axis=(1, 2), keepdims=True)
        bx2 = jnp.sum(jnp.where(selm, x2, 0.0), axis=(1, 2), keepdims=True)
        by2 = jnp.sum(jnp.where(selm, y2, 0.0), axis=(1, 2), keepdims=True)
        valid = m >= SCORE_T

        def put(r, v):
            return jnp.where((o_r == r) & (o_l == i), v, 0.0)

        acc = (acc + put(0, jnp.where(valid, bx1, 0.0))
               + put(1, jnp.where(valid, by1, 0.0))
               + put(2, jnp.where(valid, bx2, 0.0))
               + put(3, jnp.where(valid, by2, 0.0))
               + put(4, jnp.where(valid, m, 0.0)))

        ix1 = jnp.maximum(bx1, x1)
        iy1 = jnp.maximum(by1, y1)
        ix2 = jnp.minimum(bx2, x2)
        iy2 = jnp.minimum(by2, y2)
        inter = jnp.maximum(ix2 - ix1, 0.0) * jnp.maximum(iy2 - iy1, 0.0)
        ba = jnp.maximum(bx2 - bx1, 0.0) * jnp.maximum(by2 - by1, 0.0)
        union = ba + a2 - inter
        iou = jnp.where(union > 0.0, inter / jnp.maximum(union, 1e-8), 0.0)
        work = jnp.where((iou > IOU_T) | selm, NEG, work)

    o_ref[0] = acc


def _merge_body(i_ref, o_ref):
    arr = i_ref[0].reshape(NUM_CLASSES * 8, LANES)
    rr = lax.broadcasted_iota(jnp.int32, (NUM_CLASSES * 8, LANES), 0)
    ll = lax.broadcasted_iota(jnp.int32, (NUM_CLASSES * 8, LANES), 1)
    c_io = rr >> 3
    r_io = rr & 7
    score_reg = (r_io == 4) & (ll < MAXPC)
    sl = jnp.where(score_reg, arr, NEG)
    flat = c_io * MAXPC + ll

    o_r = lax.broadcasted_iota(jnp.int32, (8, LANES), 0)
    o_l = lax.broadcasted_iota(jnp.int32, (8, LANES), 1)
    acc = jnp.zeros((8, LANES), jnp.float32)
    nval = jnp.float32(0.0)

    for i in range(MAXTOT):
        m = jnp.max(sl)
        bf = jnp.min(jnp.where(sl == m, flat, jnp.int32(NUM_CLASSES * MAXPC)))
        ci = bf // MAXPC
        li = bf - ci * MAXPC
        selc = (c_io == ci) & (ll == li)
        gx1 = jnp.sum(jnp.where(selc & (r_io == 0), arr, 0.0))
        gy1 = jnp.sum(jnp.where(selc & (r_io == 1), arr, 0.0))
        gx2 = jnp.sum(jnp.where(selc & (r_io == 2), arr, 0.0))
        gy2 = jnp.sum(jnp.where(selc & (r_io == 3), arr, 0.0))
        valid = m > 0.0

        def put(r, v):
            return jnp.where((o_r == r) & (o_l == i), v, 0.0)

        acc = (acc + put(0, jnp.where(valid, gx1, 0.0))
               + put(1, jnp.where(valid, gy1, 0.0))
               + put(2, jnp.where(valid, gx2, 0.0))
               + put(3, jnp.where(valid, gy2, 0.0))
               + put(4, jnp.where(valid, m, 0.0))
               + put(5, jnp.where(valid, ci.astype(jnp.float32), 0.0)))
        nval = nval + jnp.where(valid, 1.0, 0.0)
        sl = jnp.where(selc & (r_io == 4), NEG, sl)

    acc = acc + jnp.where((o_r == 6) & (o_l == 0), nval, 0.0)
    o_ref[0] = acc


def kernel(images, predictions):
    del images
    B = predictions.shape[0]
    padn = NPAD - N_REAL
    BC = B * NUM_CLASSES

    predT = jnp.transpose(predictions[:, :, :4], (0, 2, 1))
    predT = jnp.pad(predT, ((0, 0), (0, 0), (0, padn)))
    predT = predT.reshape(B, 4, NB, LANES)
    anch = jnp.asarray(_APL)

    boxes = pl.pallas_call(
        _decode_body,
        grid=(B,),
        in_specs=[
            pl.BlockSpec((1, 4, NB, LANES), lambda b: (b, 0, 0, 0)),
            pl.BlockSpec((4, NB, LANES), lambda b: (0, 0, 0)),
        ],
        out_specs=pl.BlockSpec((1, 4, NB, LANES), lambda b: (b, 0, 0, 0)),
        out_shape=jax.ShapeDtypeStruct((B, 4, NB, LANES), jnp.float32),
    )(predT, anch)

    predp = jnp.pad(predictions, ((0, 0), (0, padn), (0, LANES - 84)),
                    constant_values=NEG)

    scm3, sb16, scol = pl.pallas_call(
        _sig_body,
        grid=(B, NPAD // SIG_CH),
        in_specs=[pl.BlockSpec((1, SIG_CH, LANES), lambda b, k: (b, k, 0))],
        out_specs=[
            pl.BlockSpec((1, LANES, SIG_CH), lambda b, k: (b, 0, k)),
            pl.BlockSpec((1, SIG_CH, LANES), lambda b, k: (b, k, 0)),
            pl.BlockSpec((1, SIG_CH, LANES), lambda b, k: (b, k, 0)),
        ],
        out_shape=[
            jax.ShapeDtypeStruct((B, LANES, NPAD), jnp.float32),
            jax.ShapeDtypeStruct((B, NPAD, LANES), jnp.bfloat16),
            jax.ShapeDtypeStruct((B, NPAD, LANES), jnp.float32),
        ],
    )(predp)

    thr = pl.pallas_call(
        _search_body,
        grid=(B,),
        in_specs=[pl.BlockSpec((1, NPAD, LANES), lambda b: (b, 0, 0))],
        out_specs=pl.BlockSpec((1, 1, LANES), lambda b: (b, 0, 0)),
        out_shape=jax.ShapeDtypeStruct((B, 1, LANES), jnp.float32),
    )(sb16)

    bcm3 = pl.pallas_call(
        _bcount_body,
        grid=(B,),
        in_specs=[
            pl.BlockSpec((1, NPAD, LANES), lambda b: (b, 0, 0)),
            pl.BlockSpec((1, 1, LANES), lambda b: (b, 0, 0)),
        ],
        out_specs=pl.BlockSpec((1, LANES, NPAD // 16), lambda b: (b, 0, 0)),
        out_shape=jax.ShapeDtypeStruct((B, LANES, NPAD // 16), jnp.float32),
    )(scol, thr)

    scm = scm3.reshape(B * LANES * NPAD)
    bcm = bcm3.reshape(B * LANES * (NPAD // 16))
    thr_flat = thr[:, 0, 4:4 + NUM_CLASSES].reshape(BC)
    brow = jnp.transpose(boxes, (0, 2, 3, 1)).reshape(B, NPAD, 4)
    brow = jnp.pad(brow, ((0, 0), (0, 0), (0, 4))).reshape(B * NPAD, 8)

    mesh = plsc.VectorSubcoreMesh(core_axis_name="c", subcore_axis_name="s")
    sc_compact = pl.kernel(
        _compact_body,
        out_type=[
            jax.ShapeDtypeStruct((BC * CAP,), jnp.float32),
            jax.ShapeDtypeStruct((4, BC * CAP), jnp.float32),
        ],
        mesh=mesh,
        compiler_params=pltpu.CompilerParams(needs_layout_passes=False,
                                             use_tc_tiling_on_sc=False),
        scratch_types=[
            pltpu.VMEM((BC,), jnp.float32),
            pltpu.VMEM((NPAD,), jnp.float32),
            pltpu.VMEM((NPAD // 16,), jnp.float32),
            pltpu.VMEM((CAP,), jnp.float32),
            pltpu.VMEM((CAP,), jnp.int32),
            pltpu.VMEM((CAP, 8), jnp.float32),
            pltpu.VMEM((4, CAP), jnp.float32),
            pltpu.SemaphoreType.DMA,
        ],
    )
    os_, op_ = sc_compact(scm, thr_flat, bcm, brow)

    cs4 = os_.reshape(B, NUM_CLASSES, CROWS, LANES)
    cb5 = op_.reshape(4, B, NUM_CLASSES, CROWS, LANES)

    per_class = pl.pallas_call(
        _cnms_body,
        grid=(B, NUM_CLASSES // CG),
        in_specs=[
            pl.BlockSpec((1, CG, CROWS, LANES), lambda b, g: (b, g, 0, 0)),
            pl.BlockSpec((4, 1, CG, CROWS, LANES),
                         lambda b, g: (0, b, g, 0, 0)),
        ],
        out_specs=pl.BlockSpec((1, CG, 8, LANES), lambda b, g: (b, g, 0, 0)),
        out_shape=jax.ShapeDtypeStruct((B, NUM_CLASSES, 8, LANES),
                                       jnp.float32),
    )(cs4, cb5)

    out3 = pl.pallas_call(
        _merge_body,
        grid=(B,),
        in_specs=[
            pl.BlockSpec((1, NUM_CLASSES, 8, LANES), lambda b: (b, 0, 0, 0)),
        ],
        out_specs=pl.BlockSpec((1, 8, LANES), lambda b: (b, 0, 0)),
        out_shape=jax.ShapeDtypeStruct((B, 8, LANES), jnp.float32),
    )(per_class)

    nb = jnp.stack(
        [out3[:, 0, :MAXTOT], out3[:, 1, :MAXTOT],
         out3[:, 2, :MAXTOT], out3[:, 3, :MAXTOT]], axis=-1)
    ns = out3[:, 4, :MAXTOT]
    nc = out3[:, 5, :MAXTOT]
    nv = out3[:, 6, 0].astype(jnp.int32)
    return nb, ns, nc, nv


# speedup vs baseline: 8.7002x; 1.0011x over previous
import functools

import numpy as np
import jax
import jax.numpy as jnp
from jax import lax
from jax.experimental import pallas as pl
from jax.experimental.pallas import tpu as pltpu
from jax.experimental.pallas import tpu_sc as plsc

NUM_CLASSES = 80
KSEL = 1000
MAXPC = 10
MAXTOT = 10
IOU_T = 0.5
SCORE_T = 0.05
LANES = 128
NEG = -1e30
CAP = 2048
CROWS = CAP // LANES
CG = 16
SIG_CH = 2048
SC_CHUNK = 8192
PAIRS_PER_TILE = 10


def _make_anchors(image_size):
    aspect_ratios = [0.5, 1.0, 2.0]
    scales = [2.0 ** 0, 2.0 ** (1.0 / 3.0), 2.0 ** (2.0 / 3.0)]
    step = int((512 - 32) / 4)
    areas = [(x * step + 32) ** 2 for x in range(5)]
    strides = [2 ** i for i in range(3, 8)]
    out = []
    for li in range(5):
        area = float(areas[li])
        dims = []
        for ratio in aspect_ratios:
            h = np.sqrt(area / ratio)
            w = area / h
            for scale in scales:
                dims.append([scale * w, scale * h])
        dims = np.array(dims, dtype=np.float32)
        fs = int(np.ceil(image_size / strides[li]))
        rx = np.arange(fs, dtype=np.float32) + 0.5
        ry = np.arange(fs, dtype=np.float32) + 0.5
        xx, yy = np.meshgrid(rx, ry)
        centers = np.stack([xx, yy], axis=-1) * strides[li]
        centers = np.tile(centers[:, :, None, :], (1, 1, 9, 1))
        dimsT = np.tile(dims[None, None, :, :], (fs, fs, 1, 1))
        anchors = np.concatenate([centers, dimsT], axis=-1).reshape(-1, 4)
        out.append(anchors)
    return np.concatenate(out, axis=0).astype(np.float32)


_ANCH = _make_anchors(512)
N_REAL = _ANCH.shape[0]
NPAD = ((N_REAL + 8 * LANES - 1) // (8 * LANES)) * (8 * LANES)
NB = NPAD // LANES

_APL = np.zeros((4, NPAD), dtype=np.float32)
_APL[:, :N_REAL] = _ANCH.T
_APL[2:, N_REAL:] = 1.0
_APL = _APL.reshape(4, NB, LANES)


def _decode_body(p_ref, a_ref, o_ref):
    tx = p_ref[0, 0]
    ty = p_ref[0, 1]
    tw = p_ref[0, 2]
    th = p_ref[0, 3]
    acx = a_ref[0]
    acy = a_ref[1]
    aw = a_ref[2]
    ah = a_ref[3]
    cx = tx * 0.1 * aw + acx
    cy = ty * 0.1 * ah + acy
    w = jnp.exp(tw * 0.2) * aw
    h = jnp.exp(th * 0.2) * ah
    o_ref[0, 0] = cx - w / 2.0
    o_ref[0, 1] = cy - h / 2.0
    o_ref[0, 2] = cx + w / 2.0
    o_ref[0, 3] = cy + h / 2.0


def _sig_body(p_ref, t_ref, h_ref, s_ref):
    x = p_ref[0]
    s = 1.0 / (1.0 + jnp.exp(-x))
    t_ref[0] = jnp.transpose(s)
    h_ref[0] = s.astype(jnp.bfloat16)
    s_ref[0] = s


def _bcount_body(s_ref, t_ref, o_ref):
    s = s_ref[0]
    t = t_ref[0]
    m = (s >= t).astype(jnp.float32)
    bc = jnp.sum(m.reshape(NPAD // 16, 16, LANES), axis=1)
    o_ref[0] = jnp.transpose(bc)


def _search_body(h_ref, t_ref):
    sb = h_ref[0]
    u = lax.bitcast_convert_type(sb, jnp.int16)

    def sbody(i, base):
        cand = base | (jnp.int32(1) << (jnp.int32(14) - i))
        cand16 = cand.astype(jnp.int16)
        cnt = jnp.sum((u >= cand16).astype(jnp.int32), axis=0, keepdims=True)
        return jnp.where(cnt >= KSEL, cand, base)

    base = lax.fori_loop(0, 15, sbody, jnp.zeros((1, LANES), jnp.int32))
    tb = jnp.maximum(base - 1, 0)
    tf = lax.bitcast_convert_type(tb.astype(jnp.int16), jnp.bfloat16)
    t_ref[0] = tf.astype(jnp.float32)


def _compact_body(scm, thr, bcm, brow, os_out, op_out,
                  thr_v, buf, bcv, cval, cidx, brows, cpl, sem):
    wid = lax.axis_index("s") * 2 + lax.axis_index("c")
    pltpu.sync_copy(thr, thr_v)

    def pair_loop(r, carry):
        p = wid * PAIRS_PER_TILE + r
        b = p // NUM_CLASSES
        c = p - b * NUM_CLASSES
        grp = (p // 16) * 16
        tv = thr_v[pl.ds(grp, 16)]
        ts = jnp.max(jnp.where(lax.iota(jnp.int32, 16) == (p - grp), tv, NEG))

        def zbody(i, c2):
            cval[pl.ds(i * 16, 16)] = jnp.zeros((16,), jnp.float32)
            cidx[pl.ds(i * 16, 16)] = jnp.zeros((16,), jnp.int32)
            return c2

        lax.fori_loop(0, CAP // 16, zbody, 0)
        row = b * LANES + 4 + c
        pltpu.sync_copy(scm.at[pl.ds(row * NPAD, NPAD)], buf)
        pltpu.sync_copy(bcm.at[pl.ds(row * (NPAD // 16), NPAD // 16)], bcv)

        def grp_body(g, off):
            cfs = bcv[pl.ds(g * 16, 16)].astype(jnp.int32)
            for q in range(16):
                cf = cfs[q]

                def write(off2=off, q=q):
                    j = g * 16 + q
                    v = buf[pl.ds(j * 16, 16)]
                    mask = v >= ts
                    co = jnp.minimum(off2, CAP - 16)
                    iv = b * NPAD + j * 16 + lax.iota(jnp.int32, 16)
                    plsc.store_compressed(cval.at[pl.ds(co, 16)], v,
                                          mask=mask)
                    plsc.store_compressed(cidx.at[pl.ds(co, 16)], iv,
                                          mask=mask)

                pl.when(cf > 0)(write)
                off = off + cf
            return off

        lax.fori_loop(0, NPAD // 256, grp_body, jnp.int32(0))

        pltpu.async_copy(brow.at[cidx], brows, sem).wait()

        def gbody(g, c2):
            rows = g * 16 + lax.iota(jnp.int32, 16)
            for j in range(4):
                vals = plsc.load_gather(
                    brows, [rows, jnp.full((16,), j, jnp.int32)])
                cpl[j, pl.ds(g * 16, 16)] = vals
            return c2

        lax.fori_loop(0, CAP // 16, gbody, 0)

        pltpu.sync_copy(cval, os_out.at[pl.ds(p * CAP, CAP)])
        for j in range(4):
            pltpu.sync_copy(cpl.at[j], op_out.at[j, pl.ds(p * CAP, CAP)])
        return carry

    lax.fori_loop(0, PAIRS_PER_TILE, pair_loop, 0)


def _cnms_body(cs_ref, bx_ref, o_ref):
    sc = cs_ref[0]
    x1 = bx_ref[0, 0]
    y1 = bx_ref[1, 0]
    x2 = bx_ref[2, 0]
    y2 = bx_ref[3, 0]
    u = lax.bitcast_convert_type(sc, jnp.int32)

    def rbody(i, base):
        cand = base | (jnp.int32(1) << (jnp.int32(30) - i))
        cnt = jnp.sum((u >= cand).astype(jnp.int32), axis=(1, 2),
                      keepdims=True)
        return jnp.where(cnt >= KSEL, cand, base)

    tstar = lax.fori_loop(0, 31, rbody, jnp.zeros((CG, 1, 1), jnp.int32))
    work = jnp.where(u >= tstar, sc, NEG)

    r_io = lax.broadcasted_iota(jnp.int32, (CG, CROWS, LANES), 1)
    l_io = lax.broadcasted_iota(jnp.int32, (CG, CROWS, LANES), 2)
    fid = r_io * LANES + l_io
    a2 = jnp.maximum(x2 - x1, 0.0) * jnp.maximum(y2 - y1, 0.0)

    o_r = lax.broadcasted_iota(jnp.int32, (CG, 8, LANES), 1)
    o_l = lax.broadcasted_iota(jnp.int32, (CG, 8, LANES), 2)
    acc = jnp.zeros((CG, 8, LANES), jnp.float32)

    for i in range(MAXPC):
        m = jnp.max(work, axis=(1, 2), keepdims=True)
        bf = jnp.min(jnp.where(work == m, fid, CAP), axis=(1, 2),
                     keepdims=True)
        selm = fid == bf
        bx1 = jnp.sum(jnp.where(selm, x1, 0.0), axis=(1, 2), keepdims=True)
        by1 = jnp.sum(jnp.where(selm, y1, 0.0), axis=(1, 2), keepdims=True)
        bx2 = jnp.sum(jnp.where(selm, x2, 0.0), axis=(1, 2), keepdims=True)
        by2 = jnp.sum(jnp.where(selm, y2, 0.0), axis=(1, 2), keepdims=True)
        valid = m >= SCORE_T

        def put(r, v):
            return jnp.where((o_r == r) & (o_l == i), v, 0.0)

        acc = (acc + put(0, jnp.where(valid, bx1, 0.0))
               + put(1, jnp.where(valid, by1, 0.0))
               + put(2, jnp.where(valid, bx2, 0.0))
               + put(3, jnp.where(valid, by2, 0.0))
               + put(4, jnp.where(valid, m, 0.0)))

        ix1 = jnp.maximum(bx1, x1)
        iy1 = jnp.maximum(by1, y1)
        ix2 = jnp.minimum(bx2, x2)
        iy2 = jnp.minimum(by2, y2)
        inter = jnp.maximum(ix2 - ix1, 0.0) * jnp.maximum(iy2 - iy1, 0.0)
        ba = jnp.maximum(bx2 - bx1, 0.0) * jnp.maximum(by2 - by1, 0.0)
        union = ba + a2 - inter
        iou = jnp.where(union > 0.0, inter / jnp.maximum(union, 1e-8), 0.0)
        work = jnp.where((iou > IOU_T) | selm, NEG, work)

    o_ref[0] = acc


def _merge_body(i_ref, o_ref):
    arr = i_ref[0].reshape(NUM_CLASSES * 8, LANES)
    rr = lax.broadcasted_iota(jnp.int32, (NUM_CLASSES * 8, LANES), 0)
    ll = lax.broadcasted_iota(jnp.int32, (NUM_CLASSES * 8, LANES), 1)
    c_io = rr >> 3
    r_io = rr & 7
    score_reg = (r_io == 4) & (ll < MAXPC)
    sl = jnp.where(score_reg, arr, NEG)
    flat = c_io * MAXPC + ll

    o_r = lax.broadcasted_iota(jnp.int32, (8, LANES), 0)
    o_l = lax.broadcasted_iota(jnp.int32, (8, LANES), 1)
    acc = jnp.zeros((8, LANES), jnp.float32)
    nval = jnp.float32(0.0)

    for i in range(MAXTOT):
        m = jnp.max(sl)
        bf = jnp.min(jnp.where(sl == m, flat, jnp.int32(NUM_CLASSES * MAXPC)))
        ci = bf // MAXPC
        li = bf - ci * MAXPC
        selc = (c_io == ci) & (ll == li)
        gx1 = jnp.sum(jnp.where(selc & (r_io == 0), arr, 0.0))
        gy1 = jnp.sum(jnp.where(selc & (r_io == 1), arr, 0.0))
        gx2 = jnp.sum(jnp.where(selc & (r_io == 2), arr, 0.0))
        gy2 = jnp.sum(jnp.where(selc & (r_io == 3), arr, 0.0))
        valid = m > 0.0

        def put(r, v):
            return jnp.where((o_r == r) & (o_l == i), v, 0.0)

        acc = (acc + put(0, jnp.where(valid, gx1, 0.0))
               + put(1, jnp.where(valid, gy1, 0.0))
               + put(2, jnp.where(valid, gx2, 0.0))
               + put(3, jnp.where(valid, gy2, 0.0))
               + put(4, jnp.where(valid, m, 0.0))
               + put(5, jnp.where(valid, ci.astype(jnp.float32), 0.0)))
        nval = nval + jnp.where(valid, 1.0, 0.0)
        sl = jnp.where(selc & (r_io == 4), NEG, sl)

    acc = acc + jnp.where((o_r == 6) & (o_l == 0), nval, 0.0)
    o_ref[0] = acc


def kernel(images, predictions):
    del images
    B = predictions.shape[0]
    padn = NPAD - N_REAL
    BC = B * NUM_CLASSES

    predT = jnp.transpose(predictions[:, :, :4], (0, 2, 1))
    predT = jnp.pad(predT, ((0, 0), (0, 0), (0, padn)))
    predT = predT.reshape(B, 4, NB, LANES)
    anch = jnp.asarray(_APL)

    boxes = pl.pallas_call(
        _decode_body,
        grid=(B,),
        in_specs=[
            pl.BlockSpec((1, 4, NB, LANES), lambda b: (b, 0, 0, 0)),
            pl.BlockSpec((4, NB, LANES), lambda b: (0, 0, 0)),
        ],
        out_specs=pl.BlockSpec((1, 4, NB, LANES), lambda b: (b, 0, 0, 0)),
        out_shape=jax.ShapeDtypeStruct((B, 4, NB, LANES), jnp.float32),
    )(predT, anch)

    predp = jnp.pad(predictions, ((0, 0), (0, padn), (0, LANES - 84)),
                    constant_values=NEG)

    scm3, sb16, scol = pl.pallas_call(
        _sig_body,
        grid=(B, NPAD // SIG_CH),
        in_specs=[pl.BlockSpec((1, SIG_CH, LANES), lambda b, k: (b, k, 0))],
        out_specs=[
            pl.BlockSpec((1, LANES, SIG_CH), lambda b, k: (b, 0, k)),
            pl.BlockSpec((1, SIG_CH, LANES), lambda b, k: (b, k, 0)),
            pl.BlockSpec((1, SIG_CH, LANES), lambda b, k: (b, k, 0)),
        ],
        out_shape=[
            jax.ShapeDtypeStruct((B, LANES, NPAD), jnp.float32),
            jax.ShapeDtypeStruct((B, NPAD, LANES), jnp.bfloat16),
            jax.ShapeDtypeStruct((B, NPAD, LANES), jnp.float32),
        ],
    )(predp)

    thr = pl.pallas_call(
        _search_body,
        grid=(B,),
        in_specs=[pl.BlockSpec((1, NPAD, LANES), lambda b: (b, 0, 0))],
        out_specs=pl.BlockSpec((1, 1, LANES), lambda b: (b, 0, 0)),
        out_shape=jax.ShapeDtypeStruct((B, 1, LANES), jnp.float32),
    )(sb16)

    bcm3 = pl.pallas_call(
        _bcount_body,
        grid=(B,),
        in_specs=[
            pl.BlockSpec((1, NPAD, LANES), lambda b: (b, 0, 0)),
            pl.BlockSpec((1, 1, LANES), lambda b: (b, 0, 0)),
        ],
        out_specs=pl.BlockSpec((1, LANES, NPAD // 16), lambda b: (b, 0, 0)),
        out_shape=jax.ShapeDtypeStruct((B, LANES, NPAD // 16), jnp.float32),
    )(scol, thr)

    scm = scm3.reshape(B * LANES * NPAD)
    bcm = bcm3.reshape(B * LANES * (NPAD // 16))
    thr_flat = thr[:, 0, 4:4 + NUM_CLASSES].reshape(BC)
    brow = jnp.transpose(boxes, (0, 2, 3, 1)).reshape(B, NPAD, 4)
    brow = jnp.pad(brow, ((0, 0), (0, 0), (0, 4))).reshape(B * NPAD, 8)

    mesh = plsc.VectorSubcoreMesh(core_axis_name="c", subcore_axis_name="s")
    sc_compact = pl.kernel(
        _compact_body,
        out_type=[
            jax.ShapeDtypeStruct((BC * CAP,), jnp.float32),
            jax.ShapeDtypeStruct((4, BC * CAP), jnp.float32),
        ],
        mesh=mesh,
        compiler_params=pltpu.CompilerParams(needs_layout_passes=False,
                                             use_tc_tiling_on_sc=False),
        scratch_types=[
            pltpu.VMEM((BC,), jnp.float32),
            pltpu.VMEM((NPAD,), jnp.float32),
            pltpu.VMEM((NPAD // 16,), jnp.float32),
            pltpu.VMEM((CAP,), jnp.float32),
            pltpu.VMEM((CAP,), jnp.int32),
            pltpu.VMEM((CAP, 8), jnp.float32),
            pltpu.VMEM((4, CAP), jnp.float32),
            pltpu.SemaphoreType.DMA,
        ],
    )
    os_, op_ = sc_compact(scm, thr_flat, bcm, brow)

    cs4 = os_.reshape(B, NUM_CLASSES, CROWS, LANES)
    cb5 = op_.reshape(4, B, NUM_CLASSES, CROWS, LANES)

    per_class = pl.pallas_call(
        _cnms_body,
        grid=(B, NUM_CLASSES // CG),
        in_specs=[
            pl.BlockSpec((1, CG, CROWS, LANES), lambda b, g: (b, g, 0, 0)),
            pl.BlockSpec((4, 1, CG, CROWS, LANES),
                         lambda b, g: (0, b, g, 0, 0)),
        ],
        out_specs=pl.BlockSpec((1, CG, 8, LANES), lambda b, g: (b, g, 0, 0)),
        out_shape=jax.ShapeDtypeStruct((B, NUM_CLASSES, 8, LANES),
                                       jnp.float32),
    )(cs4, cb5)

    out3 = pl.pallas_call(
        _merge_body,
        grid=(B,),
        in_specs=[
            pl.BlockSpec((1, NUM_CLASSES, 8, LANES), lambda b: (b, 0, 0, 0)),
        ],
        out_specs=pl.BlockSpec((1, 8, LANES), lambda b: (b, 0, 0)),
        out_shape=jax.ShapeDtypeStruct((B, 8, LANES), jnp.float32),
    )(per_class)

    nb = jnp.stack(
        [out3[:, 0, :MAXTOT], out3[:, 1, :MAXTOT],
         out3[:, 2, :MAXTOT], out3[:, 3, :MAXTOT]], axis=-1)
    ns = out3[:, 4, :MAXTOT]
    nc = out3[:, 5, :MAXTOT]
    nv = out3[:, 6, 0].astype(jnp.int32)
    return nb, ns, nc, nv


# speedup vs baseline: 14.4722x; 1.6634x over previous
import functools

import numpy as np
import jax
import jax.numpy as jnp
from jax import lax
from jax.experimental import pallas as pl
from jax.experimental.pallas import tpu as pltpu
from jax.experimental.pallas import tpu_sc as plsc

NUM_CLASSES = 80
KSEL = 1000
MAXPC = 10
MAXTOT = 10
IOU_T = 0.5
SCORE_T = 0.05
LANES = 128
NEG = -1e30
CAP = 2048
CROWS = CAP // LANES
CG = 16
SIG_CH = 2048
SC_CHUNK = 8192
PAIRS_PER_TILE = 10


def _make_anchors(image_size):
    aspect_ratios = [0.5, 1.0, 2.0]
    scales = [2.0 ** 0, 2.0 ** (1.0 / 3.0), 2.0 ** (2.0 / 3.0)]
    step = int((512 - 32) / 4)
    areas = [(x * step + 32) ** 2 for x in range(5)]
    strides = [2 ** i for i in range(3, 8)]
    out = []
    for li in range(5):
        area = float(areas[li])
        dims = []
        for ratio in aspect_ratios:
            h = np.sqrt(area / ratio)
            w = area / h
            for scale in scales:
                dims.append([scale * w, scale * h])
        dims = np.array(dims, dtype=np.float32)
        fs = int(np.ceil(image_size / strides[li]))
        rx = np.arange(fs, dtype=np.float32) + 0.5
        ry = np.arange(fs, dtype=np.float32) + 0.5
        xx, yy = np.meshgrid(rx, ry)
        centers = np.stack([xx, yy], axis=-1) * strides[li]
        centers = np.tile(centers[:, :, None, :], (1, 1, 9, 1))
        dimsT = np.tile(dims[None, None, :, :], (fs, fs, 1, 1))
        anchors = np.concatenate([centers, dimsT], axis=-1).reshape(-1, 4)
        out.append(anchors)
    return np.concatenate(out, axis=0).astype(np.float32)


_ANCH = _make_anchors(512)
N_REAL = _ANCH.shape[0]
NPAD = ((N_REAL + 8 * LANES - 1) // (8 * LANES)) * (8 * LANES)
NB = NPAD // LANES

_APL = np.zeros((4, NPAD), dtype=np.float32)
_APL[:, :N_REAL] = _ANCH.T
_APL[2:, N_REAL:] = 1.0
_APL = _APL.reshape(4, NB, LANES)


def _decode_body(p_ref, a_ref, o_ref):
    tx = p_ref[0, 0]
    ty = p_ref[0, 1]
    tw = p_ref[0, 2]
    th = p_ref[0, 3]
    acx = a_ref[0]
    acy = a_ref[1]
    aw = a_ref[2]
    ah = a_ref[3]
    cx = tx * 0.1 * aw + acx
    cy = ty * 0.1 * ah + acy
    w = jnp.exp(tw * 0.2) * aw
    h = jnp.exp(th * 0.2) * ah
    o_ref[0, 0] = cx - w / 2.0
    o_ref[0, 1] = cy - h / 2.0
    o_ref[0, 2] = cx + w / 2.0
    o_ref[0, 3] = cy + h / 2.0


def _sig_body(p_ref, t_ref, h_ref, s_ref):
    x = p_ref[0]
    s = 1.0 / (1.0 + jnp.exp(-x))
    t_ref[0] = jnp.transpose(s)
    h_ref[0] = s.astype(jnp.bfloat16)
    s_ref[0] = s


def _bcount_body(s_ref, t_ref, o_ref):
    s = s_ref[0]
    t = t_ref[0]
    m = (s >= t).astype(jnp.float32)
    bc = jnp.sum(m.reshape(NPAD // 16, 16, LANES), axis=1)
    o_ref[0] = jnp.transpose(bc)


def _search_body(h_ref, t_ref):
    sb = h_ref[0]
    u = lax.bitcast_convert_type(sb, jnp.int16)

    def sbody(i, base):
        cand = base | (jnp.int32(1) << (jnp.int32(14) - i))
        cand16 = cand.astype(jnp.int16)
        cnt = jnp.sum((u >= cand16).astype(jnp.int32), axis=0, keepdims=True)
        return jnp.where(cnt >= KSEL, cand, base)

    base = lax.fori_loop(0, 15, sbody, jnp.zeros((1, LANES), jnp.int32))
    tb = jnp.maximum(base - 1, 0)
    tf = lax.bitcast_convert_type(tb.astype(jnp.int16), jnp.bfloat16)
    t_ref[0] = tf.astype(jnp.float32)


def _compact_body(scm, thr, bcm, brow, os_out, op_out,
                  thr_v, buf, bcv, cval, cidx, brows, cpl, bspm, sem):
    core = lax.axis_index("c")
    wid = core * 16 + lax.axis_index("s")
    pltpu.sync_copy(thr, thr_v)

    @pl.when(lax.axis_index("s") == 0)
    def _():
        pltpu.sync_copy(brow.at[pl.ds(core * (2 * NPAD), 2 * NPAD)], bspm)

    plsc.subcore_barrier()

    def pair_loop(r, carry):
        p = wid * PAIRS_PER_TILE + r
        b = p // NUM_CLASSES
        c = p - b * NUM_CLASSES
        grp = (p // 16) * 16
        tv = thr_v[pl.ds(grp, 16)]
        ts = jnp.max(jnp.where(lax.iota(jnp.int32, 16) == (p - grp), tv, NEG))

        def zbody(i, c2):
            cval[pl.ds(i * 16, 16)] = jnp.zeros((16,), jnp.float32)
            cidx[pl.ds(i * 16, 16)] = jnp.zeros((16,), jnp.int32)
            return c2

        lax.fori_loop(0, CAP // 16, zbody, 0)
        row = b * LANES + 4 + c
        pltpu.sync_copy(scm.at[pl.ds(row * NPAD, NPAD)], buf)
        pltpu.sync_copy(bcm.at[pl.ds(row * (NPAD // 16), NPAD // 16)], bcv)

        def grp_body(g, off):
            cfs = bcv[pl.ds(g * 16, 16)].astype(jnp.int32)
            for q in range(16):
                cf = cfs[q]

                def write(off2=off, q=q):
                    j = g * 16 + q
                    v = buf[pl.ds(j * 16, 16)]
                    mask = v >= ts
                    co = jnp.minimum(off2, CAP - 16)
                    iv = ((b - 2 * core) * NPAD + j * 16
                          + lax.iota(jnp.int32, 16))
                    plsc.store_compressed(cval.at[pl.ds(co, 16)], v,
                                          mask=mask)
                    plsc.store_compressed(cidx.at[pl.ds(co, 16)], iv,
                                          mask=mask)

                pl.when(cf > 0)(write)
                off = off + cf
            return off

        lax.fori_loop(0, NPAD // 256, grp_body, jnp.int32(0))

        pltpu.async_copy(bspm.at[cidx], brows, sem).wait()

        def gbody(g, c2):
            rows = g * 16 + lax.iota(jnp.int32, 16)
            for j in range(4):
                vals = plsc.load_gather(
                    brows, [rows, jnp.full((16,), j, jnp.int32)])
                cpl[j, pl.ds(g * 16, 16)] = vals
            return c2

        lax.fori_loop(0, CAP // 16, gbody, 0)

        pltpu.sync_copy(cval, os_out.at[pl.ds(p * CAP, CAP)])
        for j in range(4):
            pltpu.sync_copy(cpl.at[j], op_out.at[j, pl.ds(p * CAP, CAP)])
        return carry

    lax.fori_loop(0, PAIRS_PER_TILE, pair_loop, 0)


def _cnms_body(cs_ref, bx_ref, o_ref):
    sc = cs_ref[0]
    x1 = bx_ref[0, 0]
    y1 = bx_ref[1, 0]
    x2 = bx_ref[2, 0]
    y2 = bx_ref[3, 0]
    u = lax.bitcast_convert_type(sc, jnp.int32)

    def rbody(i, base):
        cand = base | (jnp.int32(1) << (jnp.int32(30) - i))
        cnt = jnp.sum((u >= cand).astype(jnp.int32), axis=(1, 2),
                      keepdims=True)
        return jnp.where(cnt >= KSEL, cand, base)

    tstar = lax.fori_loop(0, 31, rbody, jnp.zeros((CG, 1, 1), jnp.int32))
    work = jnp.where(u >= tstar, sc, NEG)

    r_io = lax.broadcasted_iota(jnp.int32, (CG, CROWS, LANES), 1)
    l_io = lax.broadcasted_iota(jnp.int32, (CG, CROWS, LANES), 2)
    fid = r_io * LANES + l_io
    a2 = jnp.maximum(x2 - x1, 0.0) * jnp.maximum(y2 - y1, 0.0)

    o_r = lax.broadcasted_iota(jnp.int32, (CG, 8, LANES), 1)
    o_l = lax.broadcasted_iota(jnp.int32, (CG, 8, LANES), 2)
    acc = jnp.zeros((CG, 8, LANES), jnp.float32)

    for i in range(MAXPC):
        m = jnp.max(work, axis=(1, 2), keepdims=True)
        bf = jnp.min(jnp.where(work == m, fid, CAP), axis=(1, 2),
                     keepdims=True)
        selm = fid == bf
        bx1 = jnp.sum(jnp.where(selm, x1, 0.0), axis=(1, 2), keepdims=True)
        by1 = jnp.sum(jnp.where(selm, y1, 0.0), axis=(1, 2), keepdims=True)
        bx2 = jnp.sum(jnp.where(selm, x2, 0.0), axis=(1, 2), keepdims=True)
        by2 = jnp.sum(jnp.where(selm, y2, 0.0), axis=(1, 2), keepdims=True)
        valid = m >= SCORE_T

        def put(r, v):
            return jnp.where((o_r == r) & (o_l == i), v, 0.0)

        acc = (acc + put(0, jnp.where(valid, bx1, 0.0))
               + put(1, jnp.where(valid, by1, 0.0))
               + put(2, jnp.where(valid, bx2, 0.0))
               + put(3, jnp.where(valid, by2, 0.0))
               + put(4, jnp.where(valid, m, 0.0)))

        ix1 = jnp.maximum(bx1, x1)
        iy1 = jnp.maximum(by1, y1)
        ix2 = jnp.minimum(bx2, x2)
        iy2 = jnp.minimum(by2, y2)
        inter = jnp.maximum(ix2 - ix1, 0.0) * jnp.maximum(iy2 - iy1, 0.0)
        ba = jnp.maximum(bx2 - bx1, 0.0) * jnp.maximum(by2 - by1, 0.0)
        union = ba + a2 - inter
        iou = jnp.where(union > 0.0, inter / jnp.maximum(union, 1e-8), 0.0)
        work = jnp.where((iou > IOU_T) | selm, NEG, work)

    o_ref[0] = acc


def _merge_body(i_ref, o_ref):
    arr = i_ref[0].reshape(NUM_CLASSES * 8, LANES)
    rr = lax.broadcasted_iota(jnp.int32, (NUM_CLASSES * 8, LANES), 0)
    ll = lax.broadcasted_iota(jnp.int32, (NUM_CLASSES * 8, LANES), 1)
    c_io = rr >> 3
    r_io = rr & 7
    score_reg = (r_io == 4) & (ll < MAXPC)
    sl = jnp.where(score_reg, arr, NEG)
    flat = c_io * MAXPC + ll

    o_r = lax.broadcasted_iota(jnp.int32, (8, LANES), 0)
    o_l = lax.broadcasted_iota(jnp.int32, (8, LANES), 1)
    acc = jnp.zeros((8, LANES), jnp.float32)
    nval = jnp.float32(0.0)

    for i in range(MAXTOT):
        m = jnp.max(sl)
        bf = jnp.min(jnp.where(sl == m, flat, jnp.int32(NUM_CLASSES * MAXPC)))
        ci = bf // MAXPC
        li = bf - ci * MAXPC
        selc = (c_io == ci) & (ll == li)
        gx1 = jnp.sum(jnp.where(selc & (r_io == 0), arr, 0.0))
        gy1 = jnp.sum(jnp.where(selc & (r_io == 1), arr, 0.0))
        gx2 = jnp.sum(jnp.where(selc & (r_io == 2), arr, 0.0))
        gy2 = jnp.sum(jnp.where(selc & (r_io == 3), arr, 0.0))
        valid = m > 0.0

        def put(r, v):
            return jnp.where((o_r == r) & (o_l == i), v, 0.0)

        acc = (acc + put(0, jnp.where(valid, gx1, 0.0))
               + put(1, jnp.where(valid, gy1, 0.0))
               + put(2, jnp.where(valid, gx2, 0.0))
               + put(3, jnp.where(valid, gy2, 0.0))
               + put(4, jnp.where(valid, m, 0.0))
               + put(5, jnp.where(valid, ci.astype(jnp.float32), 0.0)))
        nval = nval + jnp.where(valid, 1.0, 0.0)
        sl = jnp.where(selc & (r_io == 4), NEG, sl)

    acc = acc + jnp.where((o_r == 6) & (o_l == 0), nval, 0.0)
    o_ref[0] = acc


def kernel(images, predictions):
    del images
    B = predictions.shape[0]
    padn = NPAD - N_REAL
    BC = B * NUM_CLASSES

    predT = jnp.transpose(predictions[:, :, :4], (0, 2, 1))
    predT = jnp.pad(predT, ((0, 0), (0, 0), (0, padn)))
    predT = predT.reshape(B, 4, NB, LANES)
    anch = jnp.asarray(_APL)

    boxes = pl.pallas_call(
        _decode_body,
        grid=(B,),
        in_specs=[
            pl.BlockSpec((1, 4, NB, LANES), lambda b: (b, 0, 0, 0)),
            pl.BlockSpec((4, NB, LANES), lambda b: (0, 0, 0)),
        ],
        out_specs=pl.BlockSpec((1, 4, NB, LANES), lambda b: (b, 0, 0, 0)),
        out_shape=jax.ShapeDtypeStruct((B, 4, NB, LANES), jnp.float32),
    )(predT, anch)

    predp = jnp.pad(predictions, ((0, 0), (0, padn), (0, LANES - 84)),
                    constant_values=NEG)

    scm3, sb16, scol = pl.pallas_call(
        _sig_body,
        grid=(B, NPAD // SIG_CH),
        in_specs=[pl.BlockSpec((1, SIG_CH, LANES), lambda b, k: (b, k, 0))],
        out_specs=[
            pl.BlockSpec((1, LANES, SIG_CH), lambda b, k: (b, 0, k)),
            pl.BlockSpec((1, SIG_CH, LANES), lambda b, k: (b, k, 0)),
            pl.BlockSpec((1, SIG_CH, LANES), lambda b, k: (b, k, 0)),
        ],
        out_shape=[
            jax.ShapeDtypeStruct((B, LANES, NPAD), jnp.float32),
            jax.ShapeDtypeStruct((B, NPAD, LANES), jnp.bfloat16),
            jax.ShapeDtypeStruct((B, NPAD, LANES), jnp.float32),
        ],
    )(predp)

    thr = pl.pallas_call(
        _search_body,
        grid=(B,),
        in_specs=[pl.BlockSpec((1, NPAD, LANES), lambda b: (b, 0, 0))],
        out_specs=pl.BlockSpec((1, 1, LANES), lambda b: (b, 0, 0)),
        out_shape=jax.ShapeDtypeStruct((B, 1, LANES), jnp.float32),
    )(sb16)

    bcm3 = pl.pallas_call(
        _bcount_body,
        grid=(B,),
        in_specs=[
            pl.BlockSpec((1, NPAD, LANES), lambda b: (b, 0, 0)),
            pl.BlockSpec((1, 1, LANES), lambda b: (b, 0, 0)),
        ],
        out_specs=pl.BlockSpec((1, LANES, NPAD // 16), lambda b: (b, 0, 0)),
        out_shape=jax.ShapeDtypeStruct((B, LANES, NPAD // 16), jnp.float32),
    )(scol, thr)

    scm = scm3.reshape(B * LANES * NPAD)
    bcm = bcm3.reshape(B * LANES * (NPAD // 16))
    thr_flat = thr[:, 0, 4:4 + NUM_CLASSES].reshape(BC)
    brow = jnp.transpose(boxes, (0, 2, 3, 1)).reshape(B * NPAD, 4)

    mesh = plsc.VectorSubcoreMesh(core_axis_name="c", subcore_axis_name="s")
    sc_compact = pl.kernel(
        _compact_body,
        out_type=[
            jax.ShapeDtypeStruct((BC * CAP,), jnp.float32),
            jax.ShapeDtypeStruct((4, BC * CAP), jnp.float32),
        ],
        mesh=mesh,
        compiler_params=pltpu.CompilerParams(needs_layout_passes=False,
                                             use_tc_tiling_on_sc=False),
        scratch_types=[
            pltpu.VMEM((BC,), jnp.float32),
            pltpu.VMEM((NPAD,), jnp.float32),
            pltpu.VMEM((NPAD // 16,), jnp.float32),
            pltpu.VMEM((CAP,), jnp.float32),
            pltpu.VMEM((CAP,), jnp.int32),
            pltpu.VMEM((CAP, 4), jnp.float32),
            pltpu.VMEM((4, CAP), jnp.float32),
            pltpu.VMEM_SHARED((2 * NPAD, 4), jnp.float32),
            pltpu.SemaphoreType.DMA,
        ],
    )
    os_, op_ = sc_compact(scm, thr_flat, bcm, brow)

    cs4 = os_.reshape(B, NUM_CLASSES, CROWS, LANES)
    cb5 = op_.reshape(4, B, NUM_CLASSES, CROWS, LANES)

    per_class = pl.pallas_call(
        _cnms_body,
        grid=(B, NUM_CLASSES // CG),
        in_specs=[
            pl.BlockSpec((1, CG, CROWS, LANES), lambda b, g: (b, g, 0, 0)),
            pl.BlockSpec((4, 1, CG, CROWS, LANES),
                         lambda b, g: (0, b, g, 0, 0)),
        ],
        out_specs=pl.BlockSpec((1, CG, 8, LANES), lambda b, g: (b, g, 0, 0)),
        out_shape=jax.ShapeDtypeStruct((B, NUM_CLASSES, 8, LANES),
                                       jnp.float32),
    )(cs4, cb5)

    out3 = pl.pallas_call(
        _merge_body,
        grid=(B,),
        in_specs=[
            pl.BlockSpec((1, NUM_CLASSES, 8, LANES), lambda b: (b, 0, 0, 0)),
        ],
        out_specs=pl.BlockSpec((1, 8, LANES), lambda b: (b, 0, 0)),
        out_shape=jax.ShapeDtypeStruct((B, 8, LANES), jnp.float32),
    )(per_class)

    nb = jnp.stack(
        [out3[:, 0, :MAXTOT], out3[:, 1, :MAXTOT],
         out3[:, 2, :MAXTOT], out3[:, 3, :MAXTOT]], axis=-1)
    ns = out3[:, 4, :MAXTOT]
    nc = out3[:, 5, :MAXTOT]
    nv = out3[:, 6, 0].astype(jnp.int32)
    return nb, ns, nc, nv
